# baseline (speedup 1.0000x reference)
import functools

import jax
import jax.numpy as jnp
from jax import lax
from jax.experimental import pallas as pl
from jax.experimental.pallas import tpu as pltpu

_SR = 24000.0
_NS = 48000
_T = 256
_P = 5
_TP = _NS // _T + 4
_B = 2048
_TWO_PI = 6.283185307179586


def _body(p_ref, out_ref):
    step = pl.program_id(0)

    p = p_ref[...]
    amp = p[:, 0:1]
    tau = p[:, 1:2]
    omega = p[:, 2:3]
    sigma = p[:, 3:4]
    phi = p[:, 4:5]
    gamma = p[:, 5:6]

    c = jnp.round(tau * _SR).astype(jnp.int32)
    t0 = lax.shift_right_arithmetic(c - 481, 8)
    s0 = t0 * _T

    a2 = (-0.7213475204444817 / sigma) * (1.0 / sigma)
    g2 = 0.5 * gamma
    ph_t = phi * (1.0 / _TWO_PI)
    b0 = s0.astype(jnp.float32) * (1.0 / _SR) - tau

    tf = (lax.broadcasted_iota(jnp.int32, (1, _P * _T), 1)
          .astype(jnp.float32) * (1.0 / _SR))
    dt = tf + b0
    dt2 = dt * dt
    env = jnp.exp2(dt2 * a2)
    u = (omega + g2 * dt) * dt + ph_t
    r = u - jnp.round(u)
    v = r * r
    cosv = 0.9993073635929085 + v * (
        -19.583570849792995 + v * (61.38210986681525 + v * -60.247218307967024))
    vals = env * cosv

    @pl.when(step == 0)
    def _():
        out_ref[...] = jnp.zeros_like(out_ref)

    tg = lax.broadcasted_iota(jnp.int32, (_B, _TP), 1)
    t1 = t0 + 2
    oh = jnp.where(tg == t1, amp, 0.0)
    d = lax.dot_general(
        oh, vals, (((0,), (0,)), ((), ())),
        precision=lax.Precision.DEFAULT,
        preferred_element_type=jnp.float32)
    acc = out_ref[...] + d[:, 0:_T]
    for part in range(1, _P):
        zpad = jnp.zeros((part, _T), dtype=jnp.float32)
        acc = acc + jnp.concatenate(
            [zpad, d[:-part, part * _T:(part + 1) * _T]], axis=0)
    out_ref[...] = acc


@functools.partial(jax.jit, static_argnames=())
def _render(params):
    grid = params.shape[0] // _B
    return pl.pallas_call(
        _body,
        grid=(grid,),
        in_specs=[pl.BlockSpec((_B, 6), lambda i: (i, 0))],
        out_specs=pl.BlockSpec((_TP, _T), lambda i: (0, 0)),
        out_shape=jax.ShapeDtypeStruct((_TP, _T), jnp.float32),
        compiler_params=pltpu.CompilerParams(
            dimension_semantics=("arbitrary",)),
    )(params)


def kernel(amplitude, tau, omega, sigma, phi, gamma, num_samples):
    params = jnp.stack([amplitude, tau, omega, sigma, phi, gamma], axis=1)
    padded = _render(params)
    out = padded.reshape(-1)[2 * _T:2 * _T + _NS]
    return jnp.where(jnp.arange(_NS) < num_samples, out, 0.0)
